# baseline (speedup 1.0000x reference)
import functools

import jax
import jax.numpy as jnp
from jax import lax
from jax.experimental import pallas as pl
from jax.experimental.pallas import tpu as pltpu
from jax.experimental.pallas import tpu_sc as plsc

NUM_EMB = 1_000_000
DIM = 32
NI = 16384
NJ = 50
NJP = 64

NC, NS = 2, 16
NW = NC * NS

BP = NI * NJP
ROWS_PER_W = BP // NW
CH_ROWS = 1024
N_CHUNKS = ROWS_PER_W // CH_ROWS

_mesh = plsc.VectorSubcoreMesh(core_axis_name="c", subcore_axis_name="s")


@functools.partial(
    pl.kernel,
    out_type=jax.ShapeDtypeStruct((BP, DIM), jnp.float32),
    mesh=_mesh,
    scratch_types=[
        pltpu.VMEM((ROWS_PER_W,), jnp.int32),
        pltpu.VMEM((2, CH_ROWS, DIM), jnp.float32),
        pltpu.SemaphoreType.DMA,
        pltpu.SemaphoreType.DMA,
        pltpu.SemaphoreType.DMA,
        pltpu.SemaphoreType.DMA,
    ],
    compiler_params=pltpu.CompilerParams(
        use_tc_tiling_on_sc=False, needs_layout_passes=False
    ),
)
def _emb_gather(idx_hbm, w_hbm, out_hbm, idx_v, rows_v, sg0, sg1, ss0, ss1):
    wid = lax.axis_index("s") * NC + lax.axis_index("c")
    row_base = wid * ROWS_PER_W

    pltpu.sync_copy(idx_hbm.at[pl.ds(row_base, ROWS_PER_W)], idx_v)

    def fire_gather(c, b, sem):
        pltpu.async_copy(
            w_hbm.at[idx_v.at[pl.ds(c * CH_ROWS, CH_ROWS)]], rows_v.at[b], sem
        )

    def wait_gather(b, sem):
        pltpu.make_async_copy(
            w_hbm.at[idx_v.at[pl.ds(0, CH_ROWS)]], rows_v.at[b], sem
        ).wait()

    def fire_store(c, b, sem):
        pltpu.async_copy(
            rows_v.at[b],
            out_hbm.at[pl.ds(row_base + c * CH_ROWS, CH_ROWS)],
            sem,
        )

    def wait_store(b, sem):
        pltpu.make_async_copy(
            rows_v.at[b], out_hbm.at[pl.ds(row_base, CH_ROWS)], sem
        ).wait()

    npair = N_CHUNKS // 2

    def pair(p, _):
        c0 = 2 * p

        @pl.when(p > 0)
        def _():
            wait_store(0, ss0)

        fire_gather(c0, 0, sg0)
        wait_gather(0, sg0)
        fire_store(c0, 0, ss0)

        @pl.when(p > 0)
        def _():
            wait_store(1, ss1)

        fire_gather(c0 + 1, 1, sg1)
        wait_gather(1, sg1)
        fire_store(c0 + 1, 1, ss1)
        return 0

    lax.fori_loop(0, npair, pair, 0)
    wait_store(0, ss0)
    wait_store(1, ss1)


def _transpose_body(x_ref, o_ref):
    xt = x_ref[...].T
    o_ref[...] = xt.reshape(NJP, DIM, 512)[:NJ]


def _to_native(flat):
    return pl.pallas_call(
        _transpose_body,
        grid=(NI // 512,),
        in_specs=[pl.BlockSpec((512, NJP * DIM), lambda g: (g, 0))],
        out_specs=pl.BlockSpec((NJ, DIM, 512), lambda g: (0, 0, g)),
        out_shape=jax.ShapeDtypeStruct((NJ, DIM, NI), jnp.float32),
    )(flat)


def kernel(token_ids, weight):
    w4 = lax.optimization_barrier(weight.reshape(NUM_EMB // 4, DIM * 4))
    w_rows = w4.reshape(NUM_EMB, DIM)
    tok = jnp.pad(token_ids.astype(jnp.int32), ((0, 0), (0, NJP - NJ)))
    flat_idx = tok.reshape(-1)
    rows = _emb_gather(flat_idx, w_rows)
    out_native = _to_native(rows.reshape(NI, NJP * DIM))
    return jnp.transpose(out_native, (2, 0, 1))


# speedup vs baseline: 1.0002x; 1.0002x over previous
import functools

import jax
import jax.numpy as jnp
from jax import lax
from jax.experimental import pallas as pl
from jax.experimental.pallas import tpu as pltpu
from jax.experimental.pallas import tpu_sc as plsc

NUM_EMB = 1_000_000
DIM = 32
NI = 16384
NJ = 50
NJP = 64

NC, NS = 2, 16
NW = NC * NS

BP = NI * NJP
ROWS_PER_W = BP // NW
CH_ROWS = 1024
N_CHUNKS = ROWS_PER_W // CH_ROWS

_mesh = plsc.VectorSubcoreMesh(core_axis_name="c", subcore_axis_name="s")


@functools.partial(
    pl.kernel,
    out_type=jax.ShapeDtypeStruct((BP, DIM), jnp.float32),
    mesh=_mesh,
    scratch_types=[
        pltpu.VMEM((ROWS_PER_W,), jnp.int32),
        pltpu.VMEM((2, CH_ROWS, DIM), jnp.float32),
        pltpu.SemaphoreType.DMA,
        pltpu.SemaphoreType.DMA,
        pltpu.SemaphoreType.DMA,
        pltpu.SemaphoreType.DMA,
    ],
    compiler_params=pltpu.CompilerParams(
        use_tc_tiling_on_sc=False, needs_layout_passes=False
    ),
)
def _emb_gather(idx_hbm, w_hbm, out_hbm, idx_v, rows_v, sg0, sg1, ss0, ss1):
    wid = lax.axis_index("s") * NC + lax.axis_index("c")
    row_base = wid * ROWS_PER_W

    pltpu.sync_copy(idx_hbm.at[pl.ds(row_base, ROWS_PER_W)], idx_v)

    def fire_gather(c, b, sem):
        pltpu.async_copy(
            w_hbm.at[idx_v.at[pl.ds(c * CH_ROWS, CH_ROWS)]], rows_v.at[b], sem
        )

    def wait_gather(b, sem):
        pltpu.make_async_copy(
            w_hbm.at[idx_v.at[pl.ds(0, CH_ROWS)]], rows_v.at[b], sem
        ).wait()

    def fire_store(c, b, sem):
        pltpu.async_copy(
            rows_v.at[b],
            out_hbm.at[pl.ds(row_base + c * CH_ROWS, CH_ROWS)],
            sem,
        )

    def wait_store(b, sem):
        pltpu.make_async_copy(
            rows_v.at[b], out_hbm.at[pl.ds(row_base, CH_ROWS)], sem
        ).wait()

    def chunk(c, _):
        fire_gather(c, 0, sg0)
        wait_gather(0, sg0)
        fire_store(c, 0, ss0)
        wait_store(0, ss0)
        return 0

    lax.fori_loop(0, N_CHUNKS, chunk, 0)


def _transpose_body(x_ref, o_ref):
    xt = x_ref[...].T
    o_ref[...] = xt.reshape(NJP, DIM, 512)[:NJ]


def _to_native(flat):
    return pl.pallas_call(
        _transpose_body,
        grid=(NI // 512,),
        in_specs=[pl.BlockSpec((512, NJP * DIM), lambda g: (g, 0))],
        out_specs=pl.BlockSpec((NJ, DIM, 512), lambda g: (0, 0, g)),
        out_shape=jax.ShapeDtypeStruct((NJ, DIM, NI), jnp.float32),
    )(flat)


def kernel(token_ids, weight):
    w4 = lax.optimization_barrier(weight.reshape(NUM_EMB // 4, DIM * 4))
    w_rows = w4.reshape(NUM_EMB, DIM)
    tok = jnp.pad(token_ids.astype(jnp.int32), ((0, 0), (0, NJP - NJ)))
    flat_idx = tok.reshape(-1)
    rows = _emb_gather(flat_idx, w_rows)
    out_native = _to_native(rows.reshape(NI, NJP * DIM))
    return jnp.transpose(out_native, (2, 0, 1))


# speedup vs baseline: 3.8273x; 3.8264x over previous
import functools

import jax
import jax.numpy as jnp
from jax import lax
from jax.experimental import pallas as pl
from jax.experimental.pallas import tpu as pltpu
from jax.experimental.pallas import tpu_sc as plsc

NUM_EMB = 1_000_000
DIM = 32
NI = 16384
NJ = 50
NJP = 64

NC, NS = 2, 16
NW = NC * NS

BP = NI * NJP
ROWS_PER_W = BP // NW
CH_ROWS = 1024
N_CHUNKS = ROWS_PER_W // CH_ROWS

_mesh = plsc.VectorSubcoreMesh(core_axis_name="c", subcore_axis_name="s")


@functools.partial(
    pl.kernel,
    out_type=jax.ShapeDtypeStruct((BP, DIM), jnp.float32),
    mesh=_mesh,
    scratch_types=[
        pltpu.VMEM((ROWS_PER_W,), jnp.int32),
        pltpu.VMEM((2, CH_ROWS, DIM), jnp.float32),
        pltpu.SemaphoreType.DMA,
        pltpu.SemaphoreType.DMA,
        pltpu.SemaphoreType.DMA,
        pltpu.SemaphoreType.DMA,
    ],
    compiler_params=pltpu.CompilerParams(
        use_tc_tiling_on_sc=False, needs_layout_passes=False
    ),
)
def _emb_gather(idx_hbm, w_hbm, out_hbm, idx_v, rows_v, sg0, sg1, ss0, ss1):
    wid = lax.axis_index("s") * NC + lax.axis_index("c")
    row_base = wid * ROWS_PER_W

    pltpu.sync_copy(idx_hbm.at[pl.ds(row_base, ROWS_PER_W)], idx_v)

    def fire_gather(c, b, sem):
        pltpu.async_copy(
            w_hbm.at[idx_v.at[pl.ds(c * CH_ROWS, CH_ROWS)]], rows_v.at[b], sem
        )

    def wait_gather(b, sem):
        pltpu.make_async_copy(
            w_hbm.at[idx_v.at[pl.ds(0, CH_ROWS)]], rows_v.at[b], sem
        ).wait()

    def fire_store(c, b, sem):
        pltpu.async_copy(
            rows_v.at[b],
            out_hbm.at[pl.ds(row_base + c * CH_ROWS, CH_ROWS)],
            sem,
        )

    def wait_store(b, sem):
        pltpu.make_async_copy(
            rows_v.at[b], out_hbm.at[pl.ds(row_base, CH_ROWS)], sem
        ).wait()

    def chunk(c, _):
        fire_gather(c, 0, sg0)
        wait_gather(0, sg0)
        fire_store(c, 0, ss0)
        wait_store(0, ss0)
        return 0

    lax.fori_loop(0, N_CHUNKS, chunk, 0)


def _transpose_body(x_ref, o_ref):
    xt = x_ref[...].T
    o_ref[...] = xt.reshape(NJP, DIM, 512)[:NJ]


def _to_native(flat):
    return pl.pallas_call(
        _transpose_body,
        grid=(NI // 512,),
        in_specs=[pl.BlockSpec((512, NJP * DIM), lambda g: (g, 0))],
        out_specs=pl.BlockSpec((NJ, DIM, 512), lambda g: (0, 0, g)),
        out_shape=jax.ShapeDtypeStruct((NJ, DIM, NI), jnp.float32),
    )(flat)


def kernel(token_ids, weight):
    w4 = lax.optimization_barrier(weight.reshape(NUM_EMB // 4, DIM * 4))
    w_rows = w4.reshape(NUM_EMB, DIM)
    tok32 = token_ids.astype(jnp.int32)
    tok = jnp.concatenate([tok32, tok32[:, : NJP - NJ]], axis=1)
    flat_idx = tok.reshape(-1)
    rows = _emb_gather(flat_idx, w_rows)
    out_native = _to_native(rows.reshape(NI, NJP * DIM))
    return jnp.transpose(out_native, (2, 0, 1))


# speedup vs baseline: 3.8679x; 1.0106x over previous
import functools

import jax
import jax.numpy as jnp
from jax import lax
from jax.experimental import pallas as pl
from jax.experimental.pallas import tpu as pltpu
from jax.experimental.pallas import tpu_sc as plsc

NUM_EMB = 1_000_000
DIM = 32
NI = 16384
NJ = 50
NJP = 64

NC, NS = 2, 16
NW = NC * NS

BP = NI * NJP
ROWS_PER_W = BP // NW
CH_ROWS = 1024
N_CHUNKS = ROWS_PER_W // CH_ROWS

_mesh = plsc.VectorSubcoreMesh(core_axis_name="c", subcore_axis_name="s")


@functools.partial(
    pl.kernel,
    out_type=jax.ShapeDtypeStruct((BP, DIM), jnp.float32),
    mesh=_mesh,
    scratch_types=[
        pltpu.VMEM((ROWS_PER_W,), jnp.int32),
        pltpu.VMEM((2, CH_ROWS, DIM), jnp.float32),
        pltpu.SemaphoreType.DMA,
        pltpu.SemaphoreType.DMA,
        pltpu.SemaphoreType.DMA,
        pltpu.SemaphoreType.DMA,
    ],
    compiler_params=pltpu.CompilerParams(
        use_tc_tiling_on_sc=False, needs_layout_passes=False
    ),
)
def _emb_gather(idx_hbm, w_hbm, out_hbm, idx_v, rows_v, sg0, sg1, ss0, ss1):
    wid = lax.axis_index("s") * NC + lax.axis_index("c")
    row_base = wid * ROWS_PER_W

    pltpu.sync_copy(idx_hbm.at[pl.ds(row_base, ROWS_PER_W)], idx_v)

    def fire_gather(c, b, sem):
        pltpu.async_copy(
            w_hbm.at[idx_v.at[pl.ds(c * CH_ROWS, CH_ROWS)]], rows_v.at[b], sem
        )

    def wait_gather(b, sem):
        pltpu.make_async_copy(
            w_hbm.at[idx_v.at[pl.ds(0, CH_ROWS)]], rows_v.at[b], sem
        ).wait()

    def fire_store(c, b, sem):
        pltpu.async_copy(
            rows_v.at[b],
            out_hbm.at[pl.ds(row_base + c * CH_ROWS, CH_ROWS)],
            sem,
        )

    def wait_store(b, sem):
        pltpu.make_async_copy(
            rows_v.at[b], out_hbm.at[pl.ds(row_base, CH_ROWS)], sem
        ).wait()

    def chunk(c, _):
        fire_gather(c, 0, sg0)
        wait_gather(0, sg0)
        fire_store(c, 0, ss0)
        wait_store(0, ss0)
        return 0

    lax.fori_loop(0, N_CHUNKS, chunk, 0)


def _w_transpose_body(x_ref, o_ref):
    y = x_ref[...].T.reshape(512, 4, DIM)
    for q in range(4):
        o_ref[:, DIM * q : DIM * (q + 1)] = y[:, q, :]


def _w_to_rowmajor(wt):
    return pl.pallas_call(
        _w_transpose_body,
        grid=(489,),
        in_specs=[pl.BlockSpec((DIM, 2048), lambda g: (0, g))],
        out_specs=pl.BlockSpec((512, 128), lambda g: (g, 0)),
        out_shape=jax.ShapeDtypeStruct((NUM_EMB // 4, 128), jnp.float32),
    )(wt)


def _transpose_body(x_ref, o_ref):
    xt = x_ref[...].T
    o_ref[...] = xt.reshape(NJP, DIM, 512)[:NJ]


def _to_native(flat):
    return pl.pallas_call(
        _transpose_body,
        grid=(NI // 512,),
        in_specs=[pl.BlockSpec((512, NJP * DIM), lambda g: (g, 0))],
        out_specs=pl.BlockSpec((NJ, DIM, 512), lambda g: (0, 0, g)),
        out_shape=jax.ShapeDtypeStruct((NJ, DIM, NI), jnp.float32),
    )(flat)


def kernel(token_ids, weight):
    w4 = _w_to_rowmajor(weight.T)
    w_rows = w4.reshape(NUM_EMB, DIM)
    tok32 = token_ids.astype(jnp.int32)
    tok = jnp.concatenate([tok32, tok32[:, : NJP - NJ]], axis=1)
    flat_idx = tok.reshape(-1)
    rows = _emb_gather(flat_idx, w_rows)
    out_native = _to_native(rows.reshape(NI, NJP * DIM))
    return jnp.transpose(out_native, (2, 0, 1))
